# Initial kernel scaffold; baseline (speedup 1.0000x reference)
import jax
import jax.numpy as jnp
from jax.experimental import pallas as pl


def kernel(feature, base_emb, dw_emb, ar_emb, cr_emb, strand_emb):
    raise NotImplementedError("write your pallas kernel here")


# speedup vs baseline: 50.1366x; 50.1366x over previous
import functools
import numpy as np
import jax
import jax.numpy as jnp
from jax.experimental import pallas as pl


_B, _S, _K = 1024, 50, 101
_N = _B * _S
_C = 688
_ROWS = 512


def _colmap_np():
    k = np.zeros((_C,), dtype=np.int32)
    c = 0
    for base_k, width, count in (
        (0, 8, 20),
        (40, 8, 20),
        (20, 2, 20),
        (60, 8, 20),
        (80, 8, 20),
        (100, 8, 1),
    ):
        for i in range(count):
            k[c:c + width] = base_k + i
            c += width
    assert c == _C
    return k


_COLMAP = _colmap_np()
_P = np.zeros((_K, _C), dtype=np.float32)
_P[_COLMAP, np.arange(_C)] = 1.0


def _vtab(base_emb, dw_emb, ar_emb, cr_emb, strand_emb):
    rows = []
    for j in range(3):
        rows.append(jnp.concatenate([
            jnp.tile(base_emb[j, :], 20),
            jnp.tile(dw_emb[j, :], 20),
            jnp.tile(strand_emb[j, :], 20),
            jnp.tile(ar_emb[j, :], 20),
            jnp.tile(cr_emb[j, :], 20),
            base_emb[j, :],
        ]))
    return jnp.stack(rows)


def _tc_body(feat_ref, p_ref, vtab_ref, out_ref):
    f = feat_ref[...].astype(jnp.bfloat16)
    idx = jax.lax.dot_general(
        f, p_ref[...], (((1,), (0,)), ((), ())),
        preferred_element_type=jnp.float32)
    v = vtab_ref[...]
    out_ref[...] = jnp.where(
        idx < 0.5, v[0:1, :], jnp.where(idx < 1.5, v[1:2, :], v[2:3, :]))


@jax.jit
def _tc_kernel(feat2d, vtab):
    p = jnp.asarray(_P, dtype=jnp.bfloat16)
    return pl.pallas_call(
        _tc_body,
        grid=(_N // _ROWS,),
        in_specs=[
            pl.BlockSpec((_ROWS, _K), lambda i: (i, 0)),
            pl.BlockSpec((_K, _C), lambda i: (0, 0)),
            pl.BlockSpec((3, _C), lambda i: (0, 0)),
        ],
        out_specs=pl.BlockSpec((_ROWS, _C), lambda i: (i, 0)),
        out_shape=jax.ShapeDtypeStruct((_N, _C), jnp.float32),
    )(feat2d, p, vtab)


def kernel(feature, base_emb, dw_emb, ar_emb, cr_emb, strand_emb):
    feat2d = feature.astype(jnp.int32).reshape(_N, _K)
    vtab = _vtab(base_emb, dw_emb, ar_emb, cr_emb, strand_emb)
    out = _tc_kernel(feat2d, vtab)
    return out.reshape(_B, _S, _C)
